# baseline (speedup 1.0000x reference)
import functools

import jax
import jax.numpy as jnp
from jax import lax
from jax.experimental import pallas as pl
from jax.experimental.pallas import tpu as pltpu
from jax.experimental.pallas import tpu_sc as plsc

N = 10000
NP = 10240
D = 128
E = 320000
NC = 2
NS = 16
NW = NC * NS
C = 128
NCHUNK = 79
EPW = NCHUNK * C
EPAD = NW * EPW
RPS = NP // NS
TOT = 2 * NCHUNK
KSPLIT = 134
EPS = 1e-5


def _mesh():
    return plsc.VectorSubcoreMesh(core_axis_name="c", subcore_axis_name="s")


def _sc_deg(dst4, ones_hbm):
    @functools.partial(
        pl.kernel,
        out_type=jax.ShapeDtypeStruct((NC * NP, D), jnp.float32),
        mesh=_mesh(),
        scratch_types=[
            pltpu.VMEM((1, C), jnp.int32),
            pltpu.VMEM((1, C), jnp.int32),
            pltpu.VMEM((C, D), jnp.float32),
            pltpu.VMEM_SHARED((NP, D), jnp.float32),
            pltpu.SemaphoreType.DMA,
            pltpu.SemaphoreType.DMA,
        ],
    )
    def k(dst_hbm, o_hbm, out_hbm, id0, id1, ones_v, acc, ds0, ds1):
        cid = lax.axis_index("c")
        sid = lax.axis_index("s")
        wid = sid * NC + cid
        base = sid * RPS

        pltpu.sync_copy(o_hbm, ones_v)
        for r in range(RPS // C):
            pltpu.sync_copy(ones_v, acc.at[pl.ds(base + r * C, C)])
        plsc.subcore_barrier()

        idst = (id0, id1)
        dsems = (ds0, ds1)
        for b in range(2):
            pltpu.async_copy(dst_hbm.at[wid, b], idst[b], dsems[b])

        def body(i, _):
            for b in range(2):
                jj = 2 * i + b

                @pl.when(jj < NCHUNK)
                def _slot():
                    pltpu.make_async_copy(dst_hbm.at[wid, jj], idst[b], dsems[b]).wait()
                    pltpu.sync_copy(ones_v, acc.at[idst[b].at[0]], add=True)

                    @pl.when(jj + 2 < NCHUNK)
                    def _refill():
                        pltpu.async_copy(dst_hbm.at[wid, jj + 2], idst[b], dsems[b])

            return 0

        lax.fori_loop(0, (NCHUNK + 1) // 2, body, 0)
        plsc.subcore_barrier()
        pltpu.sync_copy(acc.at[pl.ds(base, RPS)],
                        out_hbm.at[pl.ds(cid * NP + base, RPS)])

    return k(dst4, ones_hbm)


def _sc_scatter(hs, src3, dst3):
    @functools.partial(
        pl.kernel,
        out_type=jax.ShapeDtypeStruct((NC * NP, D), jnp.float32),
        mesh=_mesh(),
        scratch_types=[
            pltpu.VMEM((1, C), jnp.int32),
            pltpu.VMEM((1, C), jnp.int32),
            pltpu.VMEM((1, C), jnp.int32),
            pltpu.VMEM((1, C), jnp.int32),
            pltpu.VMEM((C, D), jnp.float32),
            pltpu.VMEM((C, D), jnp.float32),
            pltpu.VMEM_SHARED((NP, D), jnp.float32),
            pltpu.SemaphoreType.DMA,
            pltpu.SemaphoreType.DMA,
            pltpu.SemaphoreType.DMA,
            pltpu.SemaphoreType.DMA,
            pltpu.SemaphoreType.DMA,
            pltpu.SemaphoreType.DMA,
        ],
    )
    def k(hs_hbm, src_hbm, dst_hbm, out_hbm,
          is0, is1, id0, id1, rb0, rb1, acc, ss0, ss1, ds0, ds1, gs0, gs1):
        cid = lax.axis_index("c")
        sid = lax.axis_index("s")
        base = sid * RPS
        cstart = jnp.where(cid == 0, 0, KSPLIT)
        count = jnp.where(cid == 0, KSPLIT, TOT - KSPLIT)

        pltpu.sync_copy(hs_hbm.at[pl.ds(base, RPS)], acc.at[pl.ds(base, RPS)])
        plsc.subcore_barrier()

        isrc = (is0, is1)
        idst = (id0, id1)
        rbufs = (rb0, rb1)
        ssems = (ss0, ss1)
        dsems = (ds0, ds1)
        gsems = (gs0, gs1)

        for b in range(2):
            @pl.when(b < count)
            def _pro_fire():
                pltpu.async_copy(src_hbm.at[sid, cstart + b], isrc[b], ssems[b])
                pltpu.async_copy(dst_hbm.at[sid, cstart + b], idst[b], dsems[b])
        for b in range(2):
            @pl.when(b < count)
            def _pro_gather():
                pltpu.make_async_copy(src_hbm.at[sid, cstart + b], isrc[b], ssems[b]).wait()
                pltpu.async_copy(hs_hbm.at[isrc[b].at[0]], rbufs[b], gsems[b])

        def body(i, _):
            for b in range(2):
                t = 2 * i + b
                jj = cstart + t

                @pl.when(t < count)
                def _slot():
                    pltpu.make_async_copy(hs_hbm.at[isrc[b].at[0]],
                                          rbufs[b], gsems[b]).wait()

                    @pl.when(t + 2 < count)
                    def _refill_src():
                        pltpu.async_copy(src_hbm.at[sid, jj + 2], isrc[b], ssems[b])

                    pltpu.make_async_copy(dst_hbm.at[sid, jj], idst[b], dsems[b]).wait()
                    pltpu.sync_copy(rbufs[b], acc.at[idst[b].at[0]], add=True)

                    @pl.when(t + 2 < count)
                    def _next():
                        pltpu.async_copy(dst_hbm.at[sid, jj + 2], idst[b], dsems[b])
                        pltpu.make_async_copy(src_hbm.at[sid, jj + 2],
                                              isrc[b], ssems[b]).wait()
                        pltpu.async_copy(hs_hbm.at[isrc[b].at[0]], rbufs[b], gsems[b])

            return 0

        lax.fori_loop(0, (max(KSPLIT, TOT - KSPLIT) + 1) // 2, body, 0)
        plsc.subcore_barrier()
        pltpu.sync_copy(acc.at[pl.ds(base, RPS)],
                        out_hbm.at[pl.ds(cid * NP + base, RPS)])

    return k(hs, src3, dst3)


_PREC = lax.Precision.HIGHEST


def _row_mask():
    return lax.broadcasted_iota(jnp.int32, (NP, 1), 0) < N


def _tc_pre(degp, x, w0):
    def body(degp_ref, x_ref, w_ref, dinv_ref, hs_ref):
        dp = degp_ref[...]
        deg = dp[0:NP, 0:1] + dp[NP:2 * NP, 0:1] - 1.0
        dinv = 1.0 / jnp.sqrt(deg)
        dinv_b = jnp.broadcast_to(dinv, (NP, D))
        h = jnp.dot(x_ref[...], w_ref[...], preferred_element_type=jnp.float32,
                    precision=_PREC)
        dinv_ref[...] = dinv_b
        hs_ref[0:N, :] = h * dinv_b[0:N]
        hs_ref[N:NP, :] = jnp.zeros((NP - N, D), jnp.float32)

    return pl.pallas_call(
        body,
        out_shape=(jax.ShapeDtypeStruct((NP, D), jnp.float32),
                   jax.ShapeDtypeStruct((NP, D), jnp.float32)),
    )(degp, x, w0)


def _bn_relu_masked(a, g, be, mask):
    am = jnp.where(mask, a, 0.0)
    m = jnp.sum(am, axis=0, keepdims=True) * (1.0 / N)
    c = a - m
    cm = jnp.where(mask, c, 0.0)
    v = jnp.sum(cm * cm, axis=0, keepdims=True) * (1.0 / N)
    f = c / jnp.sqrt(v + EPS) * g + be
    return jnp.where(mask, jnp.maximum(f, 0.0), 0.0)


def _tc_layer(p, hs, dinv_b, b, g, be, wn, wci):
    def body(p_ref, hs_ref, dinv_ref, b_ref, g_ref, be_ref, wn_ref, wc_ref,
             hsn_ref, y_ref):
        pv = p_ref[...]
        hsv = hs_ref[...]
        dinv = dinv_ref[...]
        a = dinv * (pv[0:NP] + pv[NP:2 * NP] - hsv) + b_ref[...]
        f = _bn_relu_masked(a, g_ref[...], be_ref[...], _row_mask())
        hsn_ref[...] = jnp.dot(f, wn_ref[...], preferred_element_type=jnp.float32,
                               precision=_PREC) * dinv
        y_ref[...] = jnp.dot(f, wc_ref[...], preferred_element_type=jnp.float32,
                             precision=_PREC)

    return pl.pallas_call(
        body,
        out_shape=(jax.ShapeDtypeStruct((NP, D), jnp.float32),
                   jax.ShapeDtypeStruct((NP, 40), jnp.float32)),
    )(p, hs, dinv_b, b, g, be, wn, wci)


def _tc_final(p, hs, dinv_b, b, g, be, wc2, bc, y0, y1):
    def body(p_ref, hs_ref, dinv_ref, b_ref, g_ref, be_ref, wc_ref, bc_ref,
             y0_ref, y1_ref, out_ref):
        pv = p_ref[...]
        hsv = hs_ref[...]
        dinv = dinv_ref[...]
        a = dinv * (pv[0:NP] + pv[NP:2 * NP] - hsv) + b_ref[...]
        f = _bn_relu_masked(a, g_ref[...], be_ref[...], _row_mask())
        y2 = jnp.dot(f, wc_ref[...], preferred_element_type=jnp.float32,
                     precision=_PREC)
        y = y0_ref[...] + y1_ref[...] + y2 + bc_ref[...]
        out_ref[...] = y[0:N]

    return pl.pallas_call(
        body,
        out_shape=jax.ShapeDtypeStruct((N, 40), jnp.float32),
    )(p, hs, dinv_b, b, g, be, wc2, bc, y0, y1)


def kernel(x, edge_index, W0, b0, g0, be0, W1, b1, g1, be1, W2, b2, g2, be2,
           Wc, bc):
    pad = jnp.full((2, EPAD - E), NP - 1, dtype=edge_index.dtype)
    ei = jnp.concatenate([edge_index, pad], axis=1)
    srcS = ei[0].reshape(NS, TOT, 1, C)
    dstS = ei[1].reshape(NS, TOT, 1, C)
    dst4 = ei[1].reshape(NW, NCHUNK, 1, C)

    ones128 = jnp.ones((C, D), jnp.float32)
    degp = _sc_deg(dst4, ones128)
    dinv_b, hs1 = _tc_pre(degp, x, W0)

    b0r, g0r, be0r = b0.reshape(1, D), g0.reshape(1, D), be0.reshape(1, D)
    b1r, g1r, be1r = b1.reshape(1, D), g1.reshape(1, D), be1.reshape(1, D)
    b2r, g2r, be2r = b2.reshape(1, D), g2.reshape(1, D), be2.reshape(1, D)
    bcr = bc.reshape(1, 40)
    wc0, wc1, wc2 = Wc[0:D], Wc[D:2 * D], Wc[2 * D:3 * D]

    p1 = _sc_scatter(hs1, srcS, dstS)
    hs2, y0 = _tc_layer(p1, hs1, dinv_b, b0r, g0r, be0r, W1, wc0)
    p2 = _sc_scatter(hs2, srcS, dstS)
    hs3, y1 = _tc_layer(p2, hs2, dinv_b, b1r, g1r, be1r, W2, wc1)
    p3 = _sc_scatter(hs3, srcS, dstS)
    return _tc_final(p3, hs3, dinv_b, b2r, g2r, be2r, wc2, bcr, y0, y1)


# speedup vs baseline: 1.0003x; 1.0003x over previous
import functools

import jax
import jax.numpy as jnp
from jax import lax
from jax.experimental import pallas as pl
from jax.experimental.pallas import tpu as pltpu
from jax.experimental.pallas import tpu_sc as plsc

N = 10000
NP = 10240
D = 128
E = 320000
NC = 2
NS = 16
NW = NC * NS
C = 128
NCHUNK = 79
EPW = NCHUNK * C
EPAD = NW * EPW
RPS = NP // NS
TOT = 2 * NCHUNK
KSPLIT = 136
EPS = 1e-5


def _mesh():
    return plsc.VectorSubcoreMesh(core_axis_name="c", subcore_axis_name="s")


def _sc_deg(dst4, ones_hbm):
    @functools.partial(
        pl.kernel,
        out_type=jax.ShapeDtypeStruct((NC * NP, D), jnp.float32),
        mesh=_mesh(),
        scratch_types=[
            pltpu.VMEM((1, C), jnp.int32),
            pltpu.VMEM((1, C), jnp.int32),
            pltpu.VMEM((C, D), jnp.float32),
            pltpu.VMEM_SHARED((NP, D), jnp.float32),
            pltpu.SemaphoreType.DMA,
            pltpu.SemaphoreType.DMA,
        ],
    )
    def k(dst_hbm, o_hbm, out_hbm, id0, id1, ones_v, acc, ds0, ds1):
        cid = lax.axis_index("c")
        sid = lax.axis_index("s")
        wid = sid * NC + cid
        base = sid * RPS

        pltpu.sync_copy(o_hbm, ones_v)
        for r in range(RPS // C):
            pltpu.sync_copy(ones_v, acc.at[pl.ds(base + r * C, C)])
        plsc.subcore_barrier()

        idst = (id0, id1)
        dsems = (ds0, ds1)
        for b in range(2):
            pltpu.async_copy(dst_hbm.at[wid, b], idst[b], dsems[b])

        def body(i, _):
            for b in range(2):
                jj = 2 * i + b

                @pl.when(jj < NCHUNK)
                def _slot():
                    pltpu.make_async_copy(dst_hbm.at[wid, jj], idst[b], dsems[b]).wait()
                    pltpu.sync_copy(ones_v, acc.at[idst[b].at[0]], add=True)

                    @pl.when(jj + 2 < NCHUNK)
                    def _refill():
                        pltpu.async_copy(dst_hbm.at[wid, jj + 2], idst[b], dsems[b])

            return 0

        lax.fori_loop(0, (NCHUNK + 1) // 2, body, 0)
        plsc.subcore_barrier()
        pltpu.sync_copy(acc.at[pl.ds(base, RPS)],
                        out_hbm.at[pl.ds(cid * NP + base, RPS)])

    return k(dst4, ones_hbm)


def _sc_scatter(hs, src3, dst3):
    @functools.partial(
        pl.kernel,
        out_type=jax.ShapeDtypeStruct((NC * NP, D), jnp.float32),
        mesh=_mesh(),
        scratch_types=[
            pltpu.VMEM((1, C), jnp.int32),
            pltpu.VMEM((1, C), jnp.int32),
            pltpu.VMEM((1, C), jnp.int32),
            pltpu.VMEM((1, C), jnp.int32),
            pltpu.VMEM((C, D), jnp.float32),
            pltpu.VMEM((C, D), jnp.float32),
            pltpu.VMEM_SHARED((NP, D), jnp.float32),
            pltpu.SemaphoreType.DMA,
            pltpu.SemaphoreType.DMA,
            pltpu.SemaphoreType.DMA,
            pltpu.SemaphoreType.DMA,
            pltpu.SemaphoreType.DMA,
            pltpu.SemaphoreType.DMA,
        ],
    )
    def k(hs_hbm, src_hbm, dst_hbm, out_hbm,
          is0, is1, id0, id1, rb0, rb1, acc, ss0, ss1, ds0, ds1, gs0, gs1):
        cid = lax.axis_index("c")
        sid = lax.axis_index("s")
        base = sid * RPS
        cstart = jnp.where(cid == 0, 0, KSPLIT)
        count = jnp.where(cid == 0, KSPLIT, TOT - KSPLIT)

        pltpu.sync_copy(hs_hbm.at[pl.ds(base, RPS)], acc.at[pl.ds(base, RPS)])
        plsc.subcore_barrier()

        isrc = (is0, is1)
        idst = (id0, id1)
        rbufs = (rb0, rb1)
        ssems = (ss0, ss1)
        dsems = (ds0, ds1)
        gsems = (gs0, gs1)

        for b in range(2):
            @pl.when(b < count)
            def _pro_fire():
                pltpu.async_copy(src_hbm.at[sid, cstart + b], isrc[b], ssems[b])
                pltpu.async_copy(dst_hbm.at[sid, cstart + b], idst[b], dsems[b])
        for b in range(2):
            @pl.when(b < count)
            def _pro_gather():
                pltpu.make_async_copy(src_hbm.at[sid, cstart + b], isrc[b], ssems[b]).wait()
                pltpu.async_copy(hs_hbm.at[isrc[b].at[0]], rbufs[b], gsems[b])

        def body(i, _):
            for b in range(2):
                t = 2 * i + b
                jj = cstart + t

                @pl.when(t < count)
                def _slot():
                    pltpu.make_async_copy(hs_hbm.at[isrc[b].at[0]],
                                          rbufs[b], gsems[b]).wait()

                    @pl.when(t + 2 < count)
                    def _refill_src():
                        pltpu.async_copy(src_hbm.at[sid, jj + 2], isrc[b], ssems[b])

                    pltpu.make_async_copy(dst_hbm.at[sid, jj], idst[b], dsems[b]).wait()
                    pltpu.sync_copy(rbufs[b], acc.at[idst[b].at[0]], add=True)

                    @pl.when(t + 2 < count)
                    def _next():
                        pltpu.async_copy(dst_hbm.at[sid, jj + 2], idst[b], dsems[b])
                        pltpu.make_async_copy(src_hbm.at[sid, jj + 2],
                                              isrc[b], ssems[b]).wait()
                        pltpu.async_copy(hs_hbm.at[isrc[b].at[0]], rbufs[b], gsems[b])

            return 0

        lax.fori_loop(0, (max(KSPLIT, TOT - KSPLIT) + 1) // 2, body, 0)
        plsc.subcore_barrier()
        pltpu.sync_copy(acc.at[pl.ds(base, RPS)],
                        out_hbm.at[pl.ds(cid * NP + base, RPS)])

    return k(hs, src3, dst3)


_PREC = lax.Precision.HIGHEST


def _row_mask():
    return lax.broadcasted_iota(jnp.int32, (NP, 1), 0) < N


def _tc_pre(degp, x, w0):
    def body(degp_ref, x_ref, w_ref, dinv_ref, hs_ref):
        dp = degp_ref[...]
        deg = dp[0:NP, 0:1] + dp[NP:2 * NP, 0:1] - 1.0
        dinv = 1.0 / jnp.sqrt(deg)
        dinv_b = jnp.broadcast_to(dinv, (NP, D))
        h = jnp.dot(x_ref[...], w_ref[...], preferred_element_type=jnp.float32,
                    precision=_PREC)
        dinv_ref[...] = dinv_b
        hs_ref[0:N, :] = h * dinv_b[0:N]
        hs_ref[N:NP, :] = jnp.zeros((NP - N, D), jnp.float32)

    return pl.pallas_call(
        body,
        out_shape=(jax.ShapeDtypeStruct((NP, D), jnp.float32),
                   jax.ShapeDtypeStruct((NP, D), jnp.float32)),
    )(degp, x, w0)


def _bn_relu_masked(a, g, be, mask):
    am = jnp.where(mask, a, 0.0)
    m = jnp.sum(am, axis=0, keepdims=True) * (1.0 / N)
    c = a - m
    cm = jnp.where(mask, c, 0.0)
    v = jnp.sum(cm * cm, axis=0, keepdims=True) * (1.0 / N)
    f = c / jnp.sqrt(v + EPS) * g + be
    return jnp.where(mask, jnp.maximum(f, 0.0), 0.0)


def _tc_layer(p, hs, dinv_b, b, g, be, wn, wci):
    def body(p_ref, hs_ref, dinv_ref, b_ref, g_ref, be_ref, wn_ref, wc_ref,
             hsn_ref, y_ref):
        pv = p_ref[...]
        hsv = hs_ref[...]
        dinv = dinv_ref[...]
        a = dinv * (pv[0:NP] + pv[NP:2 * NP] - hsv) + b_ref[...]
        f = _bn_relu_masked(a, g_ref[...], be_ref[...], _row_mask())
        hsn_ref[...] = jnp.dot(f, wn_ref[...], preferred_element_type=jnp.float32,
                               precision=_PREC) * dinv
        y_ref[...] = jnp.dot(f, wc_ref[...], preferred_element_type=jnp.float32,
                             precision=_PREC)

    return pl.pallas_call(
        body,
        out_shape=(jax.ShapeDtypeStruct((NP, D), jnp.float32),
                   jax.ShapeDtypeStruct((NP, 40), jnp.float32)),
    )(p, hs, dinv_b, b, g, be, wn, wci)


def _tc_final(p, hs, dinv_b, b, g, be, wc2, bc, y0, y1):
    def body(p_ref, hs_ref, dinv_ref, b_ref, g_ref, be_ref, wc_ref, bc_ref,
             y0_ref, y1_ref, out_ref):
        pv = p_ref[...]
        hsv = hs_ref[...]
        dinv = dinv_ref[...]
        a = dinv * (pv[0:NP] + pv[NP:2 * NP] - hsv) + b_ref[...]
        f = _bn_relu_masked(a, g_ref[...], be_ref[...], _row_mask())
        y2 = jnp.dot(f, wc_ref[...], preferred_element_type=jnp.float32,
                     precision=_PREC)
        y = y0_ref[...] + y1_ref[...] + y2 + bc_ref[...]
        out_ref[...] = y[0:N]

    return pl.pallas_call(
        body,
        out_shape=jax.ShapeDtypeStruct((N, 40), jnp.float32),
    )(p, hs, dinv_b, b, g, be, wc2, bc, y0, y1)


def kernel(x, edge_index, W0, b0, g0, be0, W1, b1, g1, be1, W2, b2, g2, be2,
           Wc, bc):
    pad = jnp.full((2, EPAD - E), NP - 1, dtype=edge_index.dtype)
    ei = jnp.concatenate([edge_index, pad], axis=1)
    srcS = ei[0].reshape(NS, TOT, 1, C)
    dstS = ei[1].reshape(NS, TOT, 1, C)
    dst4 = ei[1].reshape(NW, NCHUNK, 1, C)

    ones128 = jnp.ones((C, D), jnp.float32)
    degp = _sc_deg(dst4, ones128)
    dinv_b, hs1 = _tc_pre(degp, x, W0)

    b0r, g0r, be0r = b0.reshape(1, D), g0.reshape(1, D), be0.reshape(1, D)
    b1r, g1r, be1r = b1.reshape(1, D), g1.reshape(1, D), be1.reshape(1, D)
    b2r, g2r, be2r = b2.reshape(1, D), g2.reshape(1, D), be2.reshape(1, D)
    bcr = bc.reshape(1, 40)
    wc0, wc1, wc2 = Wc[0:D], Wc[D:2 * D], Wc[2 * D:3 * D]

    p1 = _sc_scatter(hs1, srcS, dstS)
    hs2, y0 = _tc_layer(p1, hs1, dinv_b, b0r, g0r, be0r, W1, wc0)
    p2 = _sc_scatter(hs2, srcS, dstS)
    hs3, y1 = _tc_layer(p2, hs2, dinv_b, b1r, g1r, be1r, W2, wc1)
    p3 = _sc_scatter(hs3, srcS, dstS)
    return _tc_final(p3, hs3, dinv_b, b2r, g2r, be2r, wc2, bcr, y0, y1)


# speedup vs baseline: 1.0018x; 1.0015x over previous
import functools

import jax
import jax.numpy as jnp
from jax import lax
from jax.experimental import pallas as pl
from jax.experimental.pallas import tpu as pltpu
from jax.experimental.pallas import tpu_sc as plsc

N = 10000
NP = 10240
D = 128
E = 320000
NC = 2
NS = 16
NW = NC * NS
C = 128
NCHUNK = 79
EPW = NCHUNK * C
EPAD = NW * EPW
RPS = NP // NS
TOT = 2 * NCHUNK
KSPLIT = 136
EPS = 1e-5


def _mesh():
    return plsc.VectorSubcoreMesh(core_axis_name="c", subcore_axis_name="s")


def _sc_deg(dst4, ones_hbm):
    @functools.partial(
        pl.kernel,
        out_type=jax.ShapeDtypeStruct((NC * NP, D), jnp.float32),
        mesh=_mesh(),
        scratch_types=[
            pltpu.VMEM((1, C), jnp.int32),
            pltpu.VMEM((1, C), jnp.int32),
            pltpu.VMEM((1, C), jnp.int32),
            pltpu.VMEM((1, C), jnp.int32),
            pltpu.VMEM((C, D), jnp.float32),
            pltpu.VMEM_SHARED((NP, D), jnp.float32),
            pltpu.SemaphoreType.DMA,
            pltpu.SemaphoreType.DMA,
            pltpu.SemaphoreType.DMA,
            pltpu.SemaphoreType.DMA,
            pltpu.SemaphoreType.DMA,
            pltpu.SemaphoreType.DMA,
            pltpu.SemaphoreType.DMA,
            pltpu.SemaphoreType.DMA,
        ],
    )
    def k(dst_hbm, o_hbm, out_hbm, id0, id1, id2, id3, ones_v, acc,
          ds0, ds1, ds2, ds3, sc0, sc1, sc2, sc3):
        cid = lax.axis_index("c")
        sid = lax.axis_index("s")
        wid = sid * NC + cid
        base = sid * RPS

        pltpu.sync_copy(o_hbm, ones_v)
        for r in range(RPS // C):
            pltpu.sync_copy(ones_v, acc.at[pl.ds(base + r * C, C)])
        plsc.subcore_barrier()

        idst = (id0, id1, id2, id3)
        dsems = (ds0, ds1, ds2, ds3)
        scsems = (sc0, sc1, sc2, sc3)
        for b in range(4):
            pltpu.async_copy(dst_hbm.at[wid, b], idst[b], dsems[b])

        def body(i, _):
            for r in range(4):
                jj = 4 * i + r

                @pl.when(jj < NCHUNK)
                def _slot():
                    pltpu.make_async_copy(dst_hbm.at[wid, jj], idst[r], dsems[r]).wait()
                    pltpu.async_copy(ones_v, acc.at[idst[r].at[0]], scsems[r],
                                     add=True)

                    @pl.when(jj >= 2)
                    def _drain_prev():
                        rp = (r + 2) % 4
                        pltpu.make_async_copy(
                            ones_v, acc.at[idst[rp].at[0]], scsems[rp]).wait()

                        @pl.when(jj + 2 < NCHUNK)
                        def _refill():
                            pltpu.async_copy(dst_hbm.at[wid, jj + 2],
                                             idst[rp], dsems[rp])

            return 0

        lax.fori_loop(0, (NCHUNK + 3) // 4, body, 0)
        for jj in (NCHUNK - 2, NCHUNK - 1):
            r = jj % 4
            pltpu.make_async_copy(ones_v, acc.at[idst[r].at[0]], scsems[r]).wait()
        plsc.subcore_barrier()
        pltpu.sync_copy(acc.at[pl.ds(base, RPS)],
                        out_hbm.at[pl.ds(cid * NP + base, RPS)])

    return k(dst4, ones_hbm)


def _sc_scatter(hs, src3, dst3):
    @functools.partial(
        pl.kernel,
        out_type=jax.ShapeDtypeStruct((NC * NP, D), jnp.float32),
        mesh=_mesh(),
        scratch_types=[
            pltpu.VMEM((1, C), jnp.int32),
            pltpu.VMEM((1, C), jnp.int32),
            pltpu.VMEM((1, C), jnp.int32),
            pltpu.VMEM((1, C), jnp.int32),
            pltpu.VMEM((C, D), jnp.float32),
            pltpu.VMEM((C, D), jnp.float32),
            pltpu.VMEM_SHARED((NP, D), jnp.float32),
            pltpu.SemaphoreType.DMA,
            pltpu.SemaphoreType.DMA,
            pltpu.SemaphoreType.DMA,
            pltpu.SemaphoreType.DMA,
            pltpu.SemaphoreType.DMA,
            pltpu.SemaphoreType.DMA,
        ],
    )
    def k(hs_hbm, src_hbm, dst_hbm, out_hbm,
          is0, is1, id0, id1, rb0, rb1, acc, ss0, ss1, ds0, ds1, gs0, gs1):
        cid = lax.axis_index("c")
        sid = lax.axis_index("s")
        base = sid * RPS
        cstart = jnp.where(cid == 0, 0, KSPLIT)
        count = jnp.where(cid == 0, KSPLIT, TOT - KSPLIT)

        pltpu.sync_copy(hs_hbm.at[pl.ds(base, RPS)], acc.at[pl.ds(base, RPS)])
        plsc.subcore_barrier()

        isrc = (is0, is1)
        idst = (id0, id1)
        rbufs = (rb0, rb1)
        ssems = (ss0, ss1)
        dsems = (ds0, ds1)
        gsems = (gs0, gs1)

        for b in range(2):
            @pl.when(b < count)
            def _pro_fire():
                pltpu.async_copy(src_hbm.at[sid, cstart + b], isrc[b], ssems[b])
                pltpu.async_copy(dst_hbm.at[sid, cstart + b], idst[b], dsems[b])
        for b in range(2):
            @pl.when(b < count)
            def _pro_gather():
                pltpu.make_async_copy(src_hbm.at[sid, cstart + b], isrc[b], ssems[b]).wait()
                pltpu.async_copy(hs_hbm.at[isrc[b].at[0]], rbufs[b], gsems[b])

        def body(i, _):
            for b in range(2):
                t = 2 * i + b
                jj = cstart + t

                @pl.when(t < count)
                def _slot():
                    pltpu.make_async_copy(hs_hbm.at[isrc[b].at[0]],
                                          rbufs[b], gsems[b]).wait()

                    @pl.when(t + 2 < count)
                    def _refill_src():
                        pltpu.async_copy(src_hbm.at[sid, jj + 2], isrc[b], ssems[b])

                    pltpu.make_async_copy(dst_hbm.at[sid, jj], idst[b], dsems[b]).wait()
                    pltpu.sync_copy(rbufs[b], acc.at[idst[b].at[0]], add=True)

                    @pl.when(t + 2 < count)
                    def _next():
                        pltpu.async_copy(dst_hbm.at[sid, jj + 2], idst[b], dsems[b])
                        pltpu.make_async_copy(src_hbm.at[sid, jj + 2],
                                              isrc[b], ssems[b]).wait()
                        pltpu.async_copy(hs_hbm.at[isrc[b].at[0]], rbufs[b], gsems[b])

            return 0

        lax.fori_loop(0, (max(KSPLIT, TOT - KSPLIT) + 1) // 2, body, 0)
        plsc.subcore_barrier()
        pltpu.sync_copy(acc.at[pl.ds(base, RPS)],
                        out_hbm.at[pl.ds(cid * NP + base, RPS)])

    return k(hs, src3, dst3)


_PREC = lax.Precision.HIGHEST


def _row_mask():
    return lax.broadcasted_iota(jnp.int32, (NP, 1), 0) < N


def _tc_pre(degp, x, w0):
    def body(degp_ref, x_ref, w_ref, dinv_ref, hs_ref):
        dp = degp_ref[...]
        deg = dp[0:NP, 0:1] + dp[NP:2 * NP, 0:1] - 1.0
        dinv = 1.0 / jnp.sqrt(deg)
        dinv_b = jnp.broadcast_to(dinv, (NP, D))
        h = jnp.dot(x_ref[...], w_ref[...], preferred_element_type=jnp.float32,
                    precision=_PREC)
        dinv_ref[...] = dinv_b
        hs_ref[0:N, :] = h * dinv_b[0:N]
        hs_ref[N:NP, :] = jnp.zeros((NP - N, D), jnp.float32)

    return pl.pallas_call(
        body,
        out_shape=(jax.ShapeDtypeStruct((NP, D), jnp.float32),
                   jax.ShapeDtypeStruct((NP, D), jnp.float32)),
    )(degp, x, w0)


def _bn_relu_masked(a, g, be, mask):
    am = jnp.where(mask, a, 0.0)
    m = jnp.sum(am, axis=0, keepdims=True) * (1.0 / N)
    c = a - m
    cm = jnp.where(mask, c, 0.0)
    v = jnp.sum(cm * cm, axis=0, keepdims=True) * (1.0 / N)
    f = c / jnp.sqrt(v + EPS) * g + be
    return jnp.where(mask, jnp.maximum(f, 0.0), 0.0)


def _tc_layer(p, hs, dinv_b, b, g, be, wn, wci):
    def body(p_ref, hs_ref, dinv_ref, b_ref, g_ref, be_ref, wn_ref, wc_ref,
             hsn_ref, y_ref):
        pv = p_ref[...]
        hsv = hs_ref[...]
        dinv = dinv_ref[...]
        a = dinv * (pv[0:NP] + pv[NP:2 * NP] - hsv) + b_ref[...]
        f = _bn_relu_masked(a, g_ref[...], be_ref[...], _row_mask())
        hsn_ref[...] = jnp.dot(f, wn_ref[...], preferred_element_type=jnp.float32,
                               precision=_PREC) * dinv
        y_ref[...] = jnp.dot(f, wc_ref[...], preferred_element_type=jnp.float32,
                             precision=_PREC)

    return pl.pallas_call(
        body,
        out_shape=(jax.ShapeDtypeStruct((NP, D), jnp.float32),
                   jax.ShapeDtypeStruct((NP, 40), jnp.float32)),
    )(p, hs, dinv_b, b, g, be, wn, wci)


def _tc_final(p, hs, dinv_b, b, g, be, wc2, bc, y0, y1):
    def body(p_ref, hs_ref, dinv_ref, b_ref, g_ref, be_ref, wc_ref, bc_ref,
             y0_ref, y1_ref, out_ref):
        pv = p_ref[...]
        hsv = hs_ref[...]
        dinv = dinv_ref[...]
        a = dinv * (pv[0:NP] + pv[NP:2 * NP] - hsv) + b_ref[...]
        f = _bn_relu_masked(a, g_ref[...], be_ref[...], _row_mask())
        y2 = jnp.dot(f, wc_ref[...], preferred_element_type=jnp.float32,
                     precision=_PREC)
        y = y0_ref[...] + y1_ref[...] + y2 + bc_ref[...]
        out_ref[...] = y[0:N]

    return pl.pallas_call(
        body,
        out_shape=jax.ShapeDtypeStruct((N, 40), jnp.float32),
    )(p, hs, dinv_b, b, g, be, wc2, bc, y0, y1)


def kernel(x, edge_index, W0, b0, g0, be0, W1, b1, g1, be1, W2, b2, g2, be2,
           Wc, bc):
    pad = jnp.full((2, EPAD - E), NP - 1, dtype=edge_index.dtype)
    ei = jnp.concatenate([edge_index, pad], axis=1)
    srcS = ei[0].reshape(NS, TOT, 1, C)
    dstS = ei[1].reshape(NS, TOT, 1, C)
    dst4 = ei[1].reshape(NW, NCHUNK, 1, C)

    ones128 = jnp.ones((C, D), jnp.float32)
    degp = _sc_deg(dst4, ones128)
    dinv_b, hs1 = _tc_pre(degp, x, W0)

    b0r, g0r, be0r = b0.reshape(1, D), g0.reshape(1, D), be0.reshape(1, D)
    b1r, g1r, be1r = b1.reshape(1, D), g1.reshape(1, D), be1.reshape(1, D)
    b2r, g2r, be2r = b2.reshape(1, D), g2.reshape(1, D), be2.reshape(1, D)
    bcr = bc.reshape(1, 40)
    wc0, wc1, wc2 = Wc[0:D], Wc[D:2 * D], Wc[2 * D:3 * D]

    p1 = _sc_scatter(hs1, srcS, dstS)
    hs2, y0 = _tc_layer(p1, hs1, dinv_b, b0r, g0r, be0r, W1, wc0)
    p2 = _sc_scatter(hs2, srcS, dstS)
    hs3, y1 = _tc_layer(p2, hs2, dinv_b, b1r, g1r, be1r, W2, wc1)
    p3 = _sc_scatter(hs3, srcS, dstS)
    return _tc_final(p3, hs3, dinv_b, b2r, g2r, be2r, wc2, bcr, y0, y1)
